# baseline (speedup 1.0000x reference)
import jax
import jax.numpy as jnp
from jax import lax
from jax.experimental import pallas as pl
from jax.experimental.pallas import tpu as pltpu
from jax.experimental.pallas import tpu_sc as plsc

N = 24
E = 384
C = 512
NPAD = 32
LANES = 16


def _newton_rsqrt(d):
    i = plsc.bitcast(d, jnp.int32)
    y = plsc.bitcast(jnp.int32(0x5F3759DF) - (i >> 1), jnp.float32)
    for _ in range(3):
        y = y * (1.5 - 0.5 * d * y * y)
    return y


def _sc_adj_body(ei_hbm, ew_hbm, a_hbm, ei_v, ew_v, deg_v, dinv_v, a_v, sem):
    wid = lax.axis_index("s") * 2 + lax.axis_index("c")

    @pl.when(wid == 0)
    def _():
        zeros = jnp.zeros((LANES,), jnp.float32)

        in_dmas = [
            pltpu.async_copy(ei_hbm, ei_v, sem),
            pltpu.async_copy(ew_hbm, ew_v, sem),
        ]
        for r in range(N):
            a_v[r, pl.ds(0, LANES)] = zeros
            a_v[r, pl.ds(LANES, LANES)] = zeros
        for d in in_dmas:
            d.wait()

        for j in range(NPAD // LANES):
            deg_v[pl.ds(j * LANES, LANES)] = zeros
        for k in range(E // LANES):
            rv = ei_v[0, pl.ds(k * LANES, LANES)]
            wv = ew_v[pl.ds(k * LANES, LANES)]
            plsc.addupdate_scatter(deg_v, [rv], wv)

        for j in range(NPAD // LANES):
            d = deg_v[pl.ds(j * LANES, LANES)]
            dinv_v[pl.ds(j * LANES, LANES)] = jnp.where(
                d > 0.0, _newton_rsqrt(d), 0.0)

        for k in range(E // LANES):
            rv = ei_v[0, pl.ds(k * LANES, LANES)]
            cv = ei_v[1, pl.ds(k * LANES, LANES)]
            wv = ew_v[pl.ds(k * LANES, LANES)]
            dr = plsc.load_gather(dinv_v, [rv])
            dc = plsc.load_gather(dinv_v, [cv])
            plsc.addupdate_scatter(a_v, [rv, cv], -(dr * wv * dc))

        pltpu.sync_copy(a_v, a_hbm)


@jax.jit
def _sc_adj(ei, ew):
    mesh = plsc.VectorSubcoreMesh(core_axis_name="c", subcore_axis_name="s",
                                  num_cores=1)
    f = pl.kernel(
        _sc_adj_body, mesh=mesh,
        compiler_params=pltpu.CompilerParams(needs_layout_passes=False),
        out_type=jax.ShapeDtypeStruct((N, NPAD), jnp.float32),
        scratch_types=[
            pltpu.VMEM((2, E), jnp.int32),
            pltpu.VMEM((E,), jnp.float32),
            pltpu.VMEM((NPAD,), jnp.float32),
            pltpu.VMEM((NPAD,), jnp.float32),
            pltpu.VMEM((N, NPAD), jnp.float32),
            pltpu.SemaphoreType.DMA,
        ],
    )
    return f(ei, ew)


def _tc_xmm_kernel(x_ref, wz_ref, wh_ref, x0_ref, x1_ref):
    f32 = jnp.float32
    x = x_ref[:]
    x0_ref[0] = jnp.dot(x, wz_ref[0], preferred_element_type=f32)
    x1_ref[0] = jnp.dot(x, wz_ref[1], preferred_element_type=f32)
    x0_ref[1] = jnp.dot(x, wh_ref[0], preferred_element_type=f32)
    x1_ref[1] = jnp.dot(x, wh_ref[1], preferred_element_type=f32)


def _tc_gates_kernel(a_ref, x0_ref, x1_ref, u_ref, b_ref, out_ref, z_s):
    f32 = jnp.float32
    a = a_ref[:, :N]
    cg = jax.nn.relu(
        x0_ref[0] + jnp.dot(a, x1_ref[0], preferred_element_type=f32))
    t = jax.lax.dot_general(cg, u_ref[0], (((1,), (1,)), ((), ())),
                            preferred_element_type=f32) + b_ref[0]
    i = pl.program_id(0)

    @pl.when(i == 0)
    def _():
        z_s[:] = jax.nn.sigmoid(t)

    @pl.when(i == 1)
    def _():
        out_ref[:] = z_s[:] * jnp.tanh(t)


def kernel(X, edge_index, edge_weight, W_xz, W_qz, W_xr, W_qr, W_xh, W_qh,
           w_x_z, w_q_z, w_x_r, w_q_r, w_x_h, w_q_h, b_z, b_r, b_h):
    ei = edge_index.astype(jnp.int32)
    ew = edge_weight.astype(jnp.float32)
    a = _sc_adj(ei, ew)
    x0, x1 = pl.pallas_call(
        _tc_xmm_kernel,
        out_shape=[jax.ShapeDtypeStruct((2, N, C), jnp.float32)] * 2,
    )(X, W_xz, W_xh)
    u_stack = jnp.stack([w_x_z, w_x_h])
    b_stack = jnp.stack([b_z, b_h])
    return pl.pallas_call(
        _tc_gates_kernel,
        grid=(2,),
        in_specs=[
            pl.BlockSpec((N, NPAD), lambda i: (0, 0)),
            pl.BlockSpec((1, N, C), lambda i: (i, 0, 0)),
            pl.BlockSpec((1, N, C), lambda i: (i, 0, 0)),
            pl.BlockSpec((1, C, C), lambda i: (i, 0, 0)),
            pl.BlockSpec((1, N, C), lambda i: (i, 0, 0)),
        ],
        out_specs=pl.BlockSpec((N, C), lambda i: (0, 0)),
        out_shape=jax.ShapeDtypeStruct((N, C), jnp.float32),
        scratch_shapes=[pltpu.VMEM((N, C), jnp.float32)],
    )(a, x0, x1, u_stack, b_stack)


# speedup vs baseline: 1.0233x; 1.0233x over previous
import jax
import jax.numpy as jnp
from jax import lax
from jax.experimental import pallas as pl
from jax.experimental.pallas import tpu as pltpu
from jax.experimental.pallas import tpu_sc as plsc

N = 24
E = 384
C = 512
NPAD = 32
LANES = 16


def _newton_rsqrt(d):
    i = plsc.bitcast(d, jnp.int32)
    y = plsc.bitcast(jnp.int32(0x5F3759DF) - (i >> 1), jnp.float32)
    for _ in range(3):
        y = y * (1.5 - 0.5 * d * y * y)
    return y


def _sc_adj_body(ei_hbm, ew_hbm, a_hbm, ei_v, ew_v, deg_v, dinv_v, a_v, sem):
    wid = lax.axis_index("s") * 2 + lax.axis_index("c")

    @pl.when(wid == 0)
    def _():
        zeros = jnp.zeros((LANES,), jnp.float32)

        in_dmas = [
            pltpu.async_copy(ei_hbm, ei_v, sem),
            pltpu.async_copy(ew_hbm, ew_v, sem),
        ]
        for r in range(N):
            a_v[r, pl.ds(0, LANES)] = zeros
            a_v[r, pl.ds(LANES, LANES)] = zeros
        for d in in_dmas:
            d.wait()

        for j in range(NPAD // LANES):
            deg_v[pl.ds(j * LANES, LANES)] = zeros
        for k in range(E // LANES):
            rv = ei_v[0, pl.ds(k * LANES, LANES)]
            wv = ew_v[pl.ds(k * LANES, LANES)]
            plsc.addupdate_scatter(deg_v, [rv], wv)

        for j in range(NPAD // LANES):
            d = deg_v[pl.ds(j * LANES, LANES)]
            dinv_v[pl.ds(j * LANES, LANES)] = jnp.where(
                d > 0.0, _newton_rsqrt(d), 0.0)

        for k in range(E // LANES):
            rv = ei_v[0, pl.ds(k * LANES, LANES)]
            cv = ei_v[1, pl.ds(k * LANES, LANES)]
            wv = ew_v[pl.ds(k * LANES, LANES)]
            dr = plsc.load_gather(dinv_v, [rv])
            dc = plsc.load_gather(dinv_v, [cv])
            plsc.addupdate_scatter(a_v, [rv, cv], -(dr * wv * dc))

        pltpu.sync_copy(a_v, a_hbm)


@jax.jit
def _sc_adj(ei, ew):
    mesh = plsc.VectorSubcoreMesh(core_axis_name="c", subcore_axis_name="s",
                                  num_cores=1)
    f = pl.kernel(
        _sc_adj_body, mesh=mesh,
        compiler_params=pltpu.CompilerParams(needs_layout_passes=False),
        out_type=jax.ShapeDtypeStruct((N, NPAD), jnp.float32),
        scratch_types=[
            pltpu.VMEM((2, E), jnp.int32),
            pltpu.VMEM((E,), jnp.float32),
            pltpu.VMEM((NPAD,), jnp.float32),
            pltpu.VMEM((NPAD,), jnp.float32),
            pltpu.VMEM((N, NPAD), jnp.float32),
            pltpu.SemaphoreType.DMA,
        ],
    )
    return f(ei, ew)


def _tc_xmm_kernel(x_ref, wz_ref, wh_ref, x0_ref, x1_ref):
    f32 = jnp.float32
    x = x_ref[:]
    x0_ref[0] = jnp.dot(x, wz_ref[0], preferred_element_type=f32)
    x1_ref[0] = jnp.dot(x, wz_ref[1], preferred_element_type=f32)
    x0_ref[1] = jnp.dot(x, wh_ref[0], preferred_element_type=f32)
    x1_ref[1] = jnp.dot(x, wh_ref[1], preferred_element_type=f32)


def _tc_gates_kernel(a_ref, x0_ref, x1_ref, uz_ref, uh_ref,
                     bz_ref, bh_ref, out_ref):
    f32 = jnp.float32
    bf16 = jnp.bfloat16
    a = a_ref[:, :N]
    cz = jax.nn.relu(
        x0_ref[0] + jnp.dot(a, x1_ref[0], preferred_element_type=f32))
    ch = jax.nn.relu(
        x0_ref[1] + jnp.dot(a, x1_ref[1], preferred_element_type=f32))
    z = jax.nn.sigmoid(
        jax.lax.dot_general(cz.astype(bf16), uz_ref[:],
                            (((1,), (1,)), ((), ())),
                            preferred_element_type=f32) + bz_ref[:])
    ht = jnp.tanh(
        jax.lax.dot_general(ch.astype(bf16), uh_ref[:],
                            (((1,), (1,)), ((), ())),
                            preferred_element_type=f32) + bh_ref[:])
    out_ref[:] = z * ht


def kernel(X, edge_index, edge_weight, W_xz, W_qz, W_xr, W_qr, W_xh, W_qh,
           w_x_z, w_q_z, w_x_r, w_q_r, w_x_h, w_q_h, b_z, b_r, b_h):
    ei = edge_index.astype(jnp.int32)
    ew = edge_weight.astype(jnp.float32)
    a = _sc_adj(ei, ew)
    x0, x1 = pl.pallas_call(
        _tc_xmm_kernel,
        out_shape=[jax.ShapeDtypeStruct((2, N, C), jnp.float32)] * 2,
    )(X, W_xz, W_xh)
    uz_bf = w_x_z.astype(jnp.bfloat16)
    uh_bf = w_x_h.astype(jnp.bfloat16)
    return pl.pallas_call(
        _tc_gates_kernel,
        out_shape=jax.ShapeDtypeStruct((N, C), jnp.float32),
    )(a, x0, x1, uz_bf, uh_bf, b_z, b_h)


# speedup vs baseline: 1.0373x; 1.0137x over previous
import jax
import jax.numpy as jnp
from jax import lax
from jax.experimental import pallas as pl
from jax.experimental.pallas import tpu as pltpu
from jax.experimental.pallas import tpu_sc as plsc

N = 24
E = 384
C = 512
NPAD = 32
LANES = 16


def _newton_rsqrt(d):
    i = plsc.bitcast(d, jnp.int32)
    y = plsc.bitcast(jnp.int32(0x5F3759DF) - (i >> 1), jnp.float32)
    for _ in range(3):
        y = y * (1.5 - 0.5 * d * y * y)
    return y


def _sc_adj_body(ei_hbm, ew_hbm, a_hbm, ei_v, ew_v, deg_v, dinv_v, a_v, sem):
    wid = lax.axis_index("s") * 2 + lax.axis_index("c")

    @pl.when(wid == 0)
    def _():
        zeros = jnp.zeros((LANES,), jnp.float32)

        in_dmas = [
            pltpu.async_copy(ei_hbm, ei_v, sem),
            pltpu.async_copy(ew_hbm, ew_v, sem),
        ]
        for r in range(N):
            a_v[r, pl.ds(0, LANES)] = zeros
            a_v[r, pl.ds(LANES, LANES)] = zeros
        for d in in_dmas:
            d.wait()

        for j in range(NPAD // LANES):
            deg_v[pl.ds(j * LANES, LANES)] = zeros
        for k in range(E // LANES):
            rv = ei_v[0, pl.ds(k * LANES, LANES)]
            wv = ew_v[pl.ds(k * LANES, LANES)]
            plsc.addupdate_scatter(deg_v, [rv], wv)

        for j in range(NPAD // LANES):
            d = deg_v[pl.ds(j * LANES, LANES)]
            dinv_v[pl.ds(j * LANES, LANES)] = jnp.where(
                d > 0.0, _newton_rsqrt(d), 0.0)

        for k in range(E // LANES):
            rv = ei_v[0, pl.ds(k * LANES, LANES)]
            cv = ei_v[1, pl.ds(k * LANES, LANES)]
            wv = ew_v[pl.ds(k * LANES, LANES)]
            dr = plsc.load_gather(dinv_v, [rv])
            dc = plsc.load_gather(dinv_v, [cv])
            plsc.addupdate_scatter(a_v, [rv, cv], -(dr * wv * dc))

        pltpu.sync_copy(a_v, a_hbm)


@jax.jit
def _sc_adj(ei, ew):
    mesh = plsc.VectorSubcoreMesh(core_axis_name="c", subcore_axis_name="s",
                                  num_cores=1)
    f = pl.kernel(
        _sc_adj_body, mesh=mesh,
        compiler_params=pltpu.CompilerParams(needs_layout_passes=False),
        out_type=jax.ShapeDtypeStruct((N, NPAD), jnp.float32),
        scratch_types=[
            pltpu.VMEM((2, E), jnp.int32),
            pltpu.VMEM((E,), jnp.float32),
            pltpu.VMEM((NPAD,), jnp.float32),
            pltpu.VMEM((NPAD,), jnp.float32),
            pltpu.VMEM((N, NPAD), jnp.float32),
            pltpu.SemaphoreType.DMA,
        ],
    )
    return f(ei, ew)


def _tc_xmm_kernel(x_ref, wz_ref, wh_ref, x0_ref, x1_ref):
    f32 = jnp.float32
    x = x_ref[:]
    x0_ref[0] = jnp.dot(x, wz_ref[0], preferred_element_type=f32)
    x1_ref[0] = jnp.dot(x, wz_ref[1], preferred_element_type=f32)
    x0_ref[1] = jnp.dot(x, wh_ref[0], preferred_element_type=f32)
    x1_ref[1] = jnp.dot(x, wh_ref[1], preferred_element_type=f32)


def _tc_gates_kernel(a_ref, x0_ref, x1_ref, uz_ref, uh_ref,
                     bz_ref, bh_ref, out_ref):
    f32 = jnp.float32
    a = a_ref[:, :N]
    cz = jax.nn.relu(
        x0_ref[0] + jnp.dot(a, x1_ref[0], preferred_element_type=f32))
    ch = jax.nn.relu(
        x0_ref[1] + jnp.dot(a, x1_ref[1], preferred_element_type=f32))
    z = jax.nn.sigmoid(
        jax.lax.dot_general(cz, uz_ref[:], (((1,), (1,)), ((), ())),
                            preferred_element_type=f32) + bz_ref[:])
    ht = jnp.tanh(
        jax.lax.dot_general(ch, uh_ref[:], (((1,), (1,)), ((), ())),
                            preferred_element_type=f32) + bh_ref[:])
    out_ref[:] = z * ht


def kernel(X, edge_index, edge_weight, W_xz, W_qz, W_xr, W_qr, W_xh, W_qh,
           w_x_z, w_q_z, w_x_r, w_q_r, w_x_h, w_q_h, b_z, b_r, b_h):
    ei = edge_index.astype(jnp.int32)
    ew = edge_weight.astype(jnp.float32)
    a = _sc_adj(ei, ew)
    x0, x1 = pl.pallas_call(
        _tc_xmm_kernel,
        out_shape=[jax.ShapeDtypeStruct((2, N, C), jnp.float32)] * 2,
    )(X, W_xz, W_xh)
    return pl.pallas_call(
        _tc_gates_kernel,
        out_shape=jax.ShapeDtypeStruct((N, C), jnp.float32),
    )(a, x0, x1, w_x_z, w_x_h, b_z, b_h)
